# baseline (speedup 1.0000x reference)
import dataclasses
import functools

import jax
import jax.numpy as jnp
from jax import lax
from jax.experimental import pallas as pl
from jax.experimental.pallas import tpu as pltpu
from jax.experimental.pallas import tpu_sc as plsc

LANES = 16
NC, NS = 2, 16
NW = NC * NS


def kernel(feats, weights, table, bias):
    B, L = feats.shape
    V, D = table.shape
    KD = D // LANES
    RPW = B // NW
    CB = 8
    NSTEPS = RPW // CB
    IDX = CB * L
    IPG = 2 * L
    NG = IDX // IPG
    W2 = 2 * D

    feats = feats.astype(jnp.int32)
    idx2_r = (feats // 2).reshape(NW, NSTEPS, NG, IPG)
    parity = (feats % 2).astype(jnp.float32)
    wlo_r = (weights * (1.0 - parity)).reshape(NW, NSTEPS, IDX)
    whi_r = (weights * parity).reshape(NW, NSTEPS, IDX)
    table2 = table.reshape(V // 2, W2)

    mesh = plsc.VectorSubcoreMesh(core_axis_name="c", subcore_axis_name="s")

    cp = pltpu.CompilerParams()
    if "needs_layout_passes" in pltpu.CompilerParams.__dataclass_fields__:
        cp = dataclasses.replace(cp, needs_layout_passes=False)
    if "use_tc_tiling_on_sc" in pltpu.CompilerParams.__dataclass_fields__:
        cp = dataclasses.replace(cp, use_tc_tiling_on_sc=False)

    @functools.partial(
        pl.kernel,
        compiler_params=cp,
        out_type=jax.ShapeDtypeStruct((B, D), jnp.float32),
        mesh=mesh,
        scratch_types=[
            pltpu.VMEM((2, NG, IPG), jnp.int32),
            pltpu.VMEM((IDX,), jnp.float32),
            pltpu.VMEM((IDX,), jnp.float32),
            pltpu.VMEM((IDX,), jnp.float32),
            pltpu.VMEM((IDX,), jnp.float32),
            pltpu.VMEM((IDX, W2), jnp.float32),
            pltpu.VMEM((IDX, W2), jnp.float32),
            pltpu.VMEM((CB, D), jnp.float32),
            pltpu.VMEM((D,), jnp.float32),
            pltpu.SemaphoreType.DMA,
            pltpu.SemaphoreType.DMA,
        ],
    )
    def run(idx_hbm, wlo_hbm, whi_hbm, table2_hbm, bias_hbm, out_hbm,
            idx_v, wlo_v0, wlo_v1, whi_v0, whi_v1, rows_v0, rows_v1,
            out_v, bias_v, sem_g0, sem_g1):
        wid = lax.axis_index("s") * NC + lax.axis_index("c")
        gsems = (sem_g0, sem_g1)
        wlobufs = (wlo_v0, wlo_v1)
        whibufs = (whi_v0, whi_v1)
        rbufs = (rows_v0, rows_v1)

        pltpu.sync_copy(bias_hbm, bias_v)
        bchunks = tuple(bias_v[pl.ds(k * LANES, LANES)] for k in range(KD))

        def lane_bcast(vec, lane):
            dn = lax.GatherDimensionNumbers(
                offset_dims=(), collapsed_slice_dims=(0,), start_index_map=(0,))
            idx = jnp.full((LANES, 1), lane, jnp.int32)
            return lax.gather(vec, idx, dn, slice_sizes=(1,),
                              mode=lax.GatherScatterMode.PROMISE_IN_BOUNDS)

        def load_step(s, buf):
            pltpu.sync_copy(idx_hbm.at[wid, s], idx_v.at[buf])
            pltpu.sync_copy(wlo_hbm.at[wid, s], wlobufs[buf])
            pltpu.sync_copy(whi_hbm.at[wid, s], whibufs[buf])
            for g in range(NG):
                pltpu.async_copy(
                    table2_hbm.at[idx_v.at[buf, g]],
                    rbufs[buf].at[pl.ds(g * IPG, IPG)],
                    gsems[buf])

        def wait_step(buf):
            for g in range(NG):
                pltpu.make_async_copy(
                    table2_hbm.at[idx_v.at[buf, g]],
                    rbufs[buf].at[pl.ds(g * IPG, IPG)],
                    gsems[buf]).wait()

        def compute_step(s, buf):
            rows = rbufs[buf]
            wlo = wlobufs[buf]
            whi = whibufs[buf]

            @pl.loop(0, CB)
            def _(b):
                base = b * L
                starts = (0, 16, 32, 34)
                wvlo = tuple(wlo[pl.ds(base + st, LANES)] for st in starts)
                wvhi = tuple(whi[pl.ds(base + st, LANES)] for st in starts)
                acc_e = list(bchunks)
                acc_o = [jnp.zeros((LANES,), jnp.float32) for _ in range(KD)]
                for l in range(L):
                    if l < 48:
                        src, lane = divmod(l, 16)
                    else:
                        src, lane = 3, l - 34
                    ws_lo = lane_bcast(wvlo[src], lane)
                    ws_hi = lane_bcast(wvhi[src], lane)
                    tgt = acc_e if l % 2 == 0 else acc_o
                    for k in range(KD):
                        tgt[k] = (tgt[k]
                                  + ws_lo * rows[base + l,
                                                 pl.ds(k * LANES, LANES)]
                                  + ws_hi * rows[base + l,
                                                 pl.ds(D + k * LANES, LANES)])
                for k in range(KD):
                    out_v[b, pl.ds(k * LANES, LANES)] = acc_e[k] + acc_o[k]

            pltpu.sync_copy(out_v, out_hbm.at[pl.ds(wid * RPW + s * CB, CB)])

        load_step(0, 0)

        @pl.loop(0, NSTEPS // 2)
        def _(it):
            for half in range(2):
                s = it * 2 + half
                buf = half

                @pl.when(s + 1 < NSTEPS)
                def _():
                    load_step(s + 1, 1 - buf)

                wait_step(buf)
                compute_step(s, buf)

    return run(idx2_r, wlo_r, whi_r, table2, bias)


# speedup vs baseline: 2.0423x; 2.0423x over previous
import dataclasses
import functools

import jax
import jax.numpy as jnp
from jax import lax
from jax.experimental import pallas as pl
from jax.experimental.pallas import tpu as pltpu
from jax.experimental.pallas import tpu_sc as plsc

LANES = 16
NC, NS = 2, 16
NW = NC * NS


def _transpose_table(table_t):
    D, V = table_t.shape
    VB = 8064
    grid = pl.cdiv(V, VB)
    PR = VB // 2

    def body(x_ref, o_ref):
        xt = jnp.swapaxes(x_ref[...], 0, 1)
        o_ref[...] = jnp.concatenate([xt[:PR], xt[PR:]], axis=1)

    return pl.pallas_call(
        body,
        grid=(grid,),
        in_specs=[pl.BlockSpec((D, VB), lambda i: (0, i))],
        out_specs=pl.BlockSpec((PR, 2 * D), lambda i: (i, 0)),
        out_shape=jax.ShapeDtypeStruct((grid * PR, 2 * D), jnp.float32),
    )(table_t)


def kernel(feats, weights, table, bias):
    B, L = feats.shape
    V, D = table.shape
    KD = D // LANES
    RPW = B // NW
    CB = 8
    NSTEPS = RPW // CB
    IDX = CB * L
    IPG = 2 * L
    NG = IDX // IPG

    tlin = _transpose_table(jnp.swapaxes(table, 0, 1))
    VR = tlin.shape[0] * 2
    table_lin = tlin.reshape(VR, D)
    VB, PR = 8064, 4032
    f = feats.astype(jnp.int32)
    fm = f % VB
    h = (f // VB) * VB + (fm % PR) * 2 + fm // PR
    feats_r = h.reshape(NW, NSTEPS, NG, IPG)
    w_r = weights.reshape(NW, NSTEPS, IDX)

    mesh = plsc.VectorSubcoreMesh(core_axis_name="c", subcore_axis_name="s")

    cp = pltpu.CompilerParams()
    if "needs_layout_passes" in pltpu.CompilerParams.__dataclass_fields__:
        cp = dataclasses.replace(cp, needs_layout_passes=False)
    if "use_tc_tiling_on_sc" in pltpu.CompilerParams.__dataclass_fields__:
        cp = dataclasses.replace(cp, use_tc_tiling_on_sc=False)

    @functools.partial(
        pl.kernel,
        compiler_params=cp,
        out_type=jax.ShapeDtypeStruct((B, D), jnp.float32),
        mesh=mesh,
        scratch_types=[
            pltpu.VMEM((2, NG, IPG), jnp.int32),
            pltpu.VMEM((IDX,), jnp.float32),
            pltpu.VMEM((IDX,), jnp.float32),
            pltpu.VMEM((IDX, D), jnp.float32),
            pltpu.VMEM((IDX, D), jnp.float32),
            pltpu.VMEM((CB, D), jnp.float32),
            pltpu.VMEM((D,), jnp.float32),
            pltpu.SemaphoreType.DMA,
            pltpu.SemaphoreType.DMA,
        ],
    )
    def run(feats_hbm, w_hbm, table_hbm, bias_hbm, out_hbm,
            idx_v, w_v0, w_v1, rows_v0, rows_v1, out_v, bias_v,
            sem_g0, sem_g1):
        wid = lax.axis_index("s") * NC + lax.axis_index("c")
        gsems = (sem_g0, sem_g1)
        wbufs = (w_v0, w_v1)
        rbufs = (rows_v0, rows_v1)

        pltpu.sync_copy(bias_hbm, bias_v)
        bchunks = tuple(bias_v[pl.ds(k * LANES, LANES)] for k in range(KD))

        def lane_bcast(vec, lane):
            dn = lax.GatherDimensionNumbers(
                offset_dims=(), collapsed_slice_dims=(0,), start_index_map=(0,))
            idx = jnp.full((LANES, 1), lane, jnp.int32)
            return lax.gather(vec, idx, dn, slice_sizes=(1,),
                              mode=lax.GatherScatterMode.PROMISE_IN_BOUNDS)

        def load_step(s, buf):
            pltpu.sync_copy(feats_hbm.at[wid, s], idx_v.at[buf])
            pltpu.sync_copy(w_hbm.at[wid, s], wbufs[buf])
            for g in range(NG):
                pltpu.async_copy(
                    table_hbm.at[idx_v.at[buf, g]],
                    rbufs[buf].at[pl.ds(g * IPG, IPG)],
                    gsems[buf])

        def wait_step(buf):
            for g in range(NG):
                pltpu.make_async_copy(
                    table_hbm.at[idx_v.at[buf, g]],
                    rbufs[buf].at[pl.ds(g * IPG, IPG)],
                    gsems[buf]).wait()

        def compute_step(s, buf):
            rows = rbufs[buf]
            wref = wbufs[buf]

            @pl.loop(0, CB)
            def _(b):
                base = b * L
                starts = (0, 16, 32, 34)
                wv = tuple(wref[pl.ds(base + st, LANES)] for st in starts)
                acc_e = list(bchunks)
                acc_o = [jnp.zeros((LANES,), jnp.float32) for _ in range(KD)]
                for l in range(L):
                    if l < 48:
                        src, lane = divmod(l, 16)
                    else:
                        src, lane = 3, l - 34
                    wspl = lane_bcast(wv[src], lane)
                    tgt = acc_e if l % 2 == 0 else acc_o
                    for k in range(KD):
                        tgt[k] = tgt[k] + wspl * rows[base + l,
                                                      pl.ds(k * LANES, LANES)]
                for k in range(KD):
                    out_v[b, pl.ds(k * LANES, LANES)] = acc_e[k] + acc_o[k]

            pltpu.sync_copy(out_v, out_hbm.at[pl.ds(wid * RPW + s * CB, CB)])

        load_step(0, 0)

        @pl.loop(0, NSTEPS // 2)
        def _(it):
            for half in range(2):
                s = it * 2 + half
                buf = half

                @pl.when(s + 1 < NSTEPS)
                def _():
                    load_step(s + 1, 1 - buf)

                wait_step(buf)
                compute_step(s, buf)

    return run(feats_r, w_r, table_lin, bias)
